# baseline (speedup 1.0000x reference)
import functools

import jax
import jax.numpy as jnp
from jax import lax
from jax.experimental import pallas as pl
from jax.experimental.pallas import tpu as pltpu
from jax.experimental.pallas import tpu_sc as plsc

_NC = 2
_NS = 16
_NW = _NC * _NS

_CHUNK = 128


_NSLOT = 5
_UCHUNK = 1


def _gather_grid(table, idx2d, n_chunks_per_w):
    V, D = table.shape
    total_rows = idx2d.shape[0] * _CHUNK
    urows = _UCHUNK * _CHUNK
    assert n_chunks_per_w % _UCHUNK == 0
    n_units = n_chunks_per_w // _UCHUNK
    n_loop = n_units // _NSLOT
    n_epi = n_units - n_loop * _NSLOT
    mesh = plsc.VectorSubcoreMesh(core_axis_name="c", subcore_axis_name="s")

    @functools.partial(
        pl.kernel,
        out_type=jax.ShapeDtypeStruct((total_rows, D), jnp.float32),
        mesh=mesh,
        scratch_types=[
            pltpu.VMEM_SHARED((V, D), jnp.float32),
            pltpu.VMEM((_NSLOT * _UCHUNK, _CHUNK), jnp.int32),
            pltpu.VMEM((_NSLOT * urows, D), jnp.float32),
            [
                [pltpu.SemaphoreType.DMA for _ in range(_UCHUNK)]
                for _ in range(_NSLOT)
            ],
            [pltpu.SemaphoreType.DMA for _ in range(_NSLOT)],
            [pltpu.SemaphoreType.DMA for _ in range(_NSLOT)],
        ],
    )
    def run(
        table_hbm, idx_hbm, out_hbm, table_sh, idx_v, big, gsems, osems, isems
    ):
        wid = lax.axis_index("s") * _NC + lax.axis_index("c")
        chunk_base = wid * n_chunks_per_w
        row_base = chunk_base * _CHUNK
        @pl.when(lax.axis_index("s") == 0)
        def _():
            pltpu.sync_copy(table_hbm, table_sh)

        def slotbuf(s):
            return big.at[pl.ds(s * urows, urows)]

        def out_unit(u):
            return out_hbm.at[pl.ds(row_base + u * urows, urows)]

        def drain_out(s):
            pltpu.make_async_copy(slotbuf(s), out_unit(0), osems[s]).wait()

        def idx_slot(s):
            return idx_v.at[pl.ds(s * _UCHUNK, _UCHUNK)]

        def fetch_idx(u, s):
            pltpu.async_copy(
                idx_hbm.at[pl.ds(chunk_base + u * _UCHUNK, _UCHUNK)],
                idx_slot(s),
                isems[s],
            )

        def wait_idx(s):
            pltpu.make_async_copy(
                idx_hbm.at[pl.ds(0, _UCHUNK)], idx_slot(s), isems[s]
            ).wait()

        def start_gathers(s):
            descs = []
            for i in range(_UCHUNK):
                descs.append(
                    pltpu.async_copy(
                        table_sh.at[idx_v.at[s * _UCHUNK + i]],
                        big.at[pl.ds(s * urows + i * _CHUNK, _CHUNK)],
                        gsems[s][i],
                    )
                )
            return descs

        for s in range(_NSLOT):
            fetch_idx(s, s)
        plsc.subcore_barrier()

        def step(g, carry):
            descs = []
            for s in range(_NSLOT):
                u = g * _NSLOT + s

                @pl.when(g > 0)
                def _():
                    drain_out(s)

                wait_idx(s)
                descs.append(start_gathers(s))
            for s in range(_NSLOT):
                u = g * _NSLOT + s
                for d in descs[s]:
                    d.wait()
                pltpu.async_copy(slotbuf(s), out_unit(u), osems[s])

                @pl.when(u + _NSLOT < n_units)
                def _():
                    fetch_idx(u + _NSLOT, s)

            return carry

        lax.fori_loop(0, n_loop, step, 0)

        u0 = n_loop * _NSLOT
        for e in range(n_epi):
            drain_out(e)
            wait_idx(e)
            for d in start_gathers(e):
                d.wait()
            pltpu.async_copy(slotbuf(e), out_unit(u0 + e), osems[e])
        for s in range(_NSLOT):
            drain_out(s)

    return run(table, idx2d)


def kernel(order_ids, turn_ids, role_ids, turn_table):
    B, S = turn_ids.shape
    V, D = turn_table.shape
    N = B * S
    assert N % (_NW * _CHUNK) == 0
    n_chunks_per_w = N // (_NW * _CHUNK)
    idx2d = turn_ids.reshape(N // _CHUNK, _CHUNK).astype(jnp.int32)
    out = _gather_grid(turn_table, idx2d, n_chunks_per_w)
    return out.reshape(B, S, D)


# speedup vs baseline: 1.0087x; 1.0087x over previous
import functools

import jax
import jax.numpy as jnp
from jax import lax
from jax.experimental import pallas as pl
from jax.experimental.pallas import tpu as pltpu
from jax.experimental.pallas import tpu_sc as plsc

_NC = 2
_NS = 16
_NW = _NC * _NS

_CHUNK = 128


_NSLOT = 3
_UCHUNK = 2
_OSPLIT = 2


def _gather_grid(table, idx2d, n_chunks_per_w):
    V, D = table.shape
    total_rows = idx2d.shape[0] * _CHUNK
    urows = _UCHUNK * _CHUNK
    assert n_chunks_per_w % _UCHUNK == 0
    n_units = n_chunks_per_w // _UCHUNK
    n_loop = n_units // _NSLOT
    n_epi = n_units - n_loop * _NSLOT
    mesh = plsc.VectorSubcoreMesh(core_axis_name="c", subcore_axis_name="s")

    @functools.partial(
        pl.kernel,
        out_type=jax.ShapeDtypeStruct((total_rows, D), jnp.float32),
        mesh=mesh,
        scratch_types=[
            pltpu.VMEM_SHARED((V, D), jnp.float32),
            pltpu.VMEM((_NSLOT * _UCHUNK, _CHUNK), jnp.int32),
            pltpu.VMEM((_NSLOT * urows, D), jnp.float32),
            [
                [pltpu.SemaphoreType.DMA for _ in range(_UCHUNK)]
                for _ in range(_NSLOT)
            ],
            [
                [pltpu.SemaphoreType.DMA for _ in range(_OSPLIT)]
                for _ in range(_NSLOT)
            ],
            [pltpu.SemaphoreType.DMA for _ in range(_NSLOT)],
        ],
    )
    def run(
        table_hbm, idx_hbm, out_hbm, table_sh, idx_v, big, gsems, osems, isems
    ):
        wid = lax.axis_index("s") * _NC + lax.axis_index("c")
        chunk_base = wid * n_chunks_per_w
        row_base = chunk_base * _CHUNK
        @pl.when(lax.axis_index("s") == 0)
        def _():
            pltpu.sync_copy(table_hbm, table_sh)

        def slotbuf(s):
            return big.at[pl.ds(s * urows, urows)]

        def out_unit(u):
            return out_hbm.at[pl.ds(row_base + u * urows, urows)]

        prows = urows // _OSPLIT

        def issue_out(s, u):
            for p in range(_OSPLIT):
                pltpu.async_copy(
                    big.at[pl.ds(s * urows + p * prows, prows)],
                    out_hbm.at[pl.ds(row_base + u * urows + p * prows, prows)],
                    osems[s][p],
                )

        def drain_out(s):
            for p in range(_OSPLIT):
                pltpu.make_async_copy(
                    big.at[pl.ds(s * urows, prows)],
                    out_hbm.at[pl.ds(row_base, prows)],
                    osems[s][p],
                ).wait()

        def idx_slot(s):
            return idx_v.at[pl.ds(s * _UCHUNK, _UCHUNK)]

        def fetch_idx(u, s):
            pltpu.async_copy(
                idx_hbm.at[pl.ds(chunk_base + u * _UCHUNK, _UCHUNK)],
                idx_slot(s),
                isems[s],
            )

        def wait_idx(s):
            pltpu.make_async_copy(
                idx_hbm.at[pl.ds(0, _UCHUNK)], idx_slot(s), isems[s]
            ).wait()

        def start_gathers(s):
            descs = []
            for i in range(_UCHUNK):
                descs.append(
                    pltpu.async_copy(
                        table_sh.at[idx_v.at[s * _UCHUNK + i]],
                        big.at[pl.ds(s * urows + i * _CHUNK, _CHUNK)],
                        gsems[s][i],
                    )
                )
            return descs

        for s in range(_NSLOT):
            fetch_idx(s, s)
        plsc.subcore_barrier()

        def step(g, carry):
            descs = []
            for s in range(_NSLOT):
                u = g * _NSLOT + s

                @pl.when(g > 0)
                def _():
                    drain_out(s)

                wait_idx(s)
                descs.append(start_gathers(s))
            for s in range(_NSLOT):
                u = g * _NSLOT + s
                for d in descs[s]:
                    d.wait()
                issue_out(s, u)

                @pl.when(u + _NSLOT < n_units)
                def _():
                    fetch_idx(u + _NSLOT, s)

            return carry

        lax.fori_loop(0, n_loop, step, 0)

        u0 = n_loop * _NSLOT
        for e in range(n_epi):
            drain_out(e)
            wait_idx(e)
            for d in start_gathers(e):
                d.wait()
            issue_out(e, u0 + e)
        for s in range(_NSLOT):
            drain_out(s)

    return run(table, idx2d)


def kernel(order_ids, turn_ids, role_ids, turn_table):
    B, S = turn_ids.shape
    V, D = turn_table.shape
    N = B * S
    assert N % (_NW * _CHUNK) == 0
    n_chunks_per_w = N // (_NW * _CHUNK)
    idx2d = turn_ids.reshape(N // _CHUNK, _CHUNK).astype(jnp.int32)
    out = _gather_grid(turn_table, idx2d, n_chunks_per_w)
    return out.reshape(B, S, D)
